# baseline (speedup 1.0000x reference)
import functools

import jax
import jax.numpy as jnp
from jax import lax
from jax.experimental import pallas as pl
from jax.experimental.pallas import tpu as pltpu
from jax.experimental.pallas import tpu_sc as plsc

N = 16384
C = 1000
NUM_CORES = 2
NUM_SUBCORES = 16
NW = NUM_CORES * NUM_SUBCORES
ROWS_PER_W = N // NW
CHUNK = 128
NCHUNK = ROWS_PER_W // CHUNK
LANES = 16


def _sc_body(prob_hbm, tgt_hbm, rew_hbm, out_hbm,
             tgt_v, rew_v, idx_v, val_v, acc_v, sem):
    wid = lax.axis_index("s") * NUM_CORES + lax.axis_index("c")
    base = wid * ROWS_PER_W

    pltpu.sync_copy(tgt_hbm.at[pl.ds(base, ROWS_PER_W)], tgt_v)
    pltpu.sync_copy(rew_hbm.at[pl.ds(base, ROWS_PER_W)], rew_v)

    lane_iota = lax.broadcasted_iota(jnp.int32, (LANES,), 0)

    for j in range(NCHUNK):
        row_j = idx_v.at[j]
        for i in range(CHUNK // LANES):
            off = j * CHUNK + i * LANES
            t = tgt_v[pl.ds(off, LANES)]
            rows = (base + off) + lane_iota
            row_j[pl.ds(i * LANES, LANES)] = rows * C + t

    copies = [
        pltpu.async_copy(prob_hbm.at[idx_v.at[j]], val_v.at[j], sem)
        for j in range(NCHUNK)
    ]
    for c in copies:
        c.wait()

    acc = jnp.zeros((LANES,), jnp.float32)
    for j in range(NCHUNK):
        val_j = val_v.at[j]
        for i in range(CHUNK // LANES):
            off = j * CHUNK + i * LANES
            acc = acc + val_j[pl.ds(i * LANES, LANES)] * rew_v[pl.ds(off, LANES)]
    acc_v[...] = acc
    pltpu.sync_copy(acc_v, out_hbm.at[wid])


@jax.jit
def _sc_gather_loss(prob_flat, target_i32, reward):
    mesh = plsc.VectorSubcoreMesh(
        core_axis_name="c", subcore_axis_name="s",
        num_cores=NUM_CORES, num_subcores=NUM_SUBCORES)
    run = pl.kernel(
        _sc_body,
        out_type=jax.ShapeDtypeStruct((NW, LANES), jnp.float32),
        mesh=mesh,
        scratch_types=[
            pltpu.VMEM((ROWS_PER_W,), jnp.int32),
            pltpu.VMEM((ROWS_PER_W,), jnp.float32),
            pltpu.VMEM((NCHUNK, CHUNK), jnp.int32),
            pltpu.VMEM((NCHUNK, CHUNK), jnp.float32),
            pltpu.VMEM((LANES,), jnp.float32),
            pltpu.SemaphoreType.DMA,
        ],
    )
    return run(prob_flat, target_i32, reward)


def kernel(prob, target, reward):
    partials = _sc_gather_loss(
        prob.reshape(-1), target.astype(jnp.int32), reward)
    return -jnp.sum(partials)


# speedup vs baseline: 1.2466x; 1.2466x over previous
import jax
import jax.numpy as jnp
from jax import lax
from jax.experimental import pallas as pl
from jax.experimental.pallas import tpu as pltpu
from jax.experimental.pallas import tpu_sc as plsc

N = 16384
C = 1000
NUM_CORES = 2
NUM_SUBCORES = 16
NW = NUM_CORES * NUM_SUBCORES
RPW = N // NW
LANES = 16
NTILE = 7
TAIL = C - NTILE * 128
PAGE = 112
CAP = 5 * PAGE
NGRP = RPW // LANES
PGRP = PAGE // LANES


def _sc_body(prob_hbm, tail_hbm, tgt_hbm, rew_hbm, out_hbm,
             tgt_v, rew_v, aidx_v, aorig_v, val_v, tval_v, acc_v,
             sem_s, sem_e):
    wid = lax.axis_index("s") * NUM_CORES + lax.axis_index("c")
    base = wid * RPW

    pltpu.sync_copy(tgt_hbm.at[pl.ds(base, RPW)], tgt_v.at[pl.ds(0, RPW)])
    pltpu.sync_copy(rew_hbm.at[pl.ds(base, RPW)], rew_v.at[pl.ds(0, RPW)])

    iota = lax.broadcasted_iota(jnp.int32, (LANES,), 0)
    rew_v[pl.ds(RPW, LANES)] = jnp.zeros((LANES,), jnp.float32)
    tgt_v[pl.ds(RPW, LANES)] = iota * 0

    def fill_body(g, _):
        o = g * LANES
        aidx_v[pl.ds(o, LANES)] = ((o + iota) * 64) & (N - 1)
        aorig_v[pl.ds(o, LANES)] = iota * 0 + RPW
        return 0

    lax.fori_loop(0, 8 * CAP // LANES, fill_body, 0)

    def cnt_body(g, cnts):
        tile = tgt_v[pl.ds(g * LANES, LANES)] >> 7
        return tuple(
            cnts[k] + plsc.all_reduce_population_count(tile == k)
            for k in range(8)
        )

    zero_i = jnp.zeros((LANES,), jnp.int32)
    cnts = lax.fori_loop(0, NGRP, cnt_body, (zero_i,) * 8)

    pages_s = [jnp.max(((cnts[k] + (PAGE - 1)) * 9363) >> 20)
               for k in range(8)]

    def scat_body(g, curs):
        off = g * LANES
        c = tgt_v[pl.ds(off, LANES)]
        r = (base + off) + iota
        tile = c >> 7
        new_curs = []
        for k in range(8):
            m = tile == k
            m01 = jnp.where(m, 1, 0)
            pos = curs[k] + plsc.cumsum(m01) - m01
            value = r if k < NTILE else r * TAIL + (c - NTILE * 128)
            plsc.store_scatter(aidx_v, [pos], value, mask=m)
            plsc.store_scatter(aorig_v, [pos], off + iota, mask=m)
            new_curs.append(curs[k] + plsc.all_reduce_population_count(m))
        return tuple(new_curs)

    lax.fori_loop(0, NGRP, scat_body,
                  tuple(zero_i + k * CAP for k in range(8)))

    strip_cps = [
        pltpu.async_copy(
            prob_hbm.at[plsc.Indices(aidx_v.at[pl.ds(k * CAP, PAGE)]),
                        pl.ds(k * 128, 128)],
            val_v.at[k], sem_s)
        for k in range(NTILE)
    ]
    tail_cp = pltpu.async_copy(
        tail_hbm.at[aidx_v.at[pl.ds(NTILE * CAP, PAGE)]], tval_v, sem_e)
    for cp in strip_cps:
        cp.wait()
    tail_cp.wait()

    def extract_strip(k_static, page_base, a):
        def grp(g, a2):
            q = pl.multiple_of(page_base + g * LANES, LANES)
            orig = aorig_v[pl.ds(q, LANES)]
            lane = plsc.load_gather(tgt_v, [orig]) & 127
            v = plsc.load_gather(val_v.at[k_static], [g * LANES + iota, lane])
            return a2 + v * plsc.load_gather(rew_v, [orig])
        return lax.fori_loop(0, PGRP, grp, a)

    def tail_extract(page_base, a):
        def grp(g, a2):
            q = pl.multiple_of(page_base + g * LANES, LANES)
            orig = aorig_v[pl.ds(q, LANES)]
            v = tval_v[pl.ds(g * LANES, LANES)]
            return a2 + v * plsc.load_gather(rew_v, [orig])
        return lax.fori_loop(0, PGRP, grp, a)

    acc = jnp.zeros((LANES,), jnp.float32)
    for k in range(NTILE):
        acc = extract_strip(k, k * CAP, acc)
    acc = tail_extract(NTILE * CAP, acc)

    for k in range(NTILE):
        def ovf_body(p, a, k=k):
            pb = pl.multiple_of(k * CAP + p * PAGE, LANES)
            cp = pltpu.async_copy(
                prob_hbm.at[plsc.Indices(aidx_v.at[pl.ds(pb, PAGE)]),
                            pl.ds(k * 128, 128)],
                val_v.at[k], sem_s)
            cp.wait()
            return extract_strip(k, pb, a)
        acc = lax.fori_loop(1, pages_s[k], ovf_body, acc)

    def tovf_body(p, a):
        pb = pl.multiple_of(NTILE * CAP + p * PAGE, LANES)
        cp = pltpu.async_copy(
            tail_hbm.at[aidx_v.at[pl.ds(pb, PAGE)]], tval_v, sem_e)
        cp.wait()
        return tail_extract(pb, a)

    acc = lax.fori_loop(1, pages_s[NTILE], tovf_body, acc)

    acc_v[...] = acc
    pltpu.sync_copy(acc_v, out_hbm.at[wid])


@jax.jit
def _sc_gather_loss(prob, tail_flat, target_i32, reward):
    mesh = plsc.VectorSubcoreMesh(
        core_axis_name="c", subcore_axis_name="s",
        num_cores=NUM_CORES, num_subcores=NUM_SUBCORES)
    run = pl.kernel(
        _sc_body,
        out_type=jax.ShapeDtypeStruct((NW, LANES), jnp.float32),
        mesh=mesh,
        scratch_types=[
            pltpu.VMEM((RPW + LANES,), jnp.int32),
            pltpu.VMEM((RPW + LANES,), jnp.float32),
            pltpu.VMEM((8 * CAP,), jnp.int32),
            pltpu.VMEM((8 * CAP,), jnp.int32),
            pltpu.VMEM((NTILE, PAGE, 128), jnp.float32),
            pltpu.VMEM((PAGE,), jnp.float32),
            pltpu.VMEM((LANES,), jnp.float32),
            pltpu.SemaphoreType.DMA,
            pltpu.SemaphoreType.DMA,
        ],
        compiler_params=pltpu.CompilerParams(needs_layout_passes=False),
    )
    return run(prob, tail_flat, target_i32, reward)


def kernel(prob, target, reward):
    tail_flat = prob[:, NTILE * 128:].reshape(-1)
    partials = _sc_gather_loss(
        prob, tail_flat, target.astype(jnp.int32), reward)
    return -jnp.sum(partials)
